# initial kernel scaffold (unmeasured)
import jax
import jax.numpy as jnp
from jax import lax
from jax.experimental import pallas as pl
from jax.experimental.pallas import tpu as pltpu


def kernel(
    x,
):
    def body(*refs):
        pass

    out_shape = jax.ShapeDtypeStruct(..., jnp.float32)
    return pl.pallas_call(body, out_shape=out_shape)(...)


# baseline (device time: 3729115 ns/iter reference)
import jax
import jax.numpy as jnp
from jax import lax
from jax.experimental import pallas as pl
from jax.experimental.pallas import tpu as pltpu

N_DEV = 8


def kernel(x):
    m, n = x.shape

    def body(x_ref, out_ref, copy_sem, send_sems, recv_sems):
        my = lax.axis_index("i")
        left = lax.rem(my + N_DEV - 1, N_DEV)
        right = lax.rem(my + 1, N_DEV)

        barrier_sem = pltpu.get_barrier_semaphore()
        pl.semaphore_signal(
            barrier_sem, inc=1, device_id=(left,),
            device_id_type=pl.DeviceIdType.MESH,
        )
        pl.semaphore_signal(
            barrier_sem, inc=1, device_id=(right,),
            device_id_type=pl.DeviceIdType.MESH,
        )
        pl.semaphore_wait(barrier_sem, 2)

        local = pltpu.make_async_copy(
            x_ref, out_ref.at[pl.ds(my * m, m)], copy_sem
        )
        local.start()
        local.wait()

        for h in range(N_DEV - 1):
            origin = lax.rem(my - h + N_DEV, N_DEV)
            rdma = pltpu.make_async_remote_copy(
                src_ref=out_ref.at[pl.ds(origin * m, m)],
                dst_ref=out_ref.at[pl.ds(origin * m, m)],
                send_sem=send_sems.at[h],
                recv_sem=recv_sems.at[h],
                device_id=(right,),
                device_id_type=pl.DeviceIdType.MESH,
            )
            rdma.start()
            rdma.wait()

    return pl.pallas_call(
        body,
        out_shape=jax.ShapeDtypeStruct((N_DEV * m, n), x.dtype),
        in_specs=[pl.BlockSpec(memory_space=pltpu.MemorySpace.HBM)],
        out_specs=pl.BlockSpec(memory_space=pltpu.MemorySpace.HBM),
        scratch_shapes=[
            pltpu.SemaphoreType.DMA,
            pltpu.SemaphoreType.DMA((N_DEV - 1,)),
            pltpu.SemaphoreType.DMA((N_DEV - 1,)),
        ],
        compiler_params=pltpu.CompilerParams(collective_id=0),
    )(x)


# device time: 1453189 ns/iter; 2.5662x vs baseline; 2.5662x over previous
import jax
import jax.numpy as jnp
from jax import lax
from jax.experimental import pallas as pl
from jax.experimental.pallas import tpu as pltpu

N_DEV = 8


def kernel(x):
    m, n = x.shape

    mh = m // 2

    def body(x_ref, out_ref, copy_sem, send_r, recv_r, send_l, recv_l):
        my = lax.axis_index("i")
        left = lax.rem(my + N_DEV - 1, N_DEV)
        right = lax.rem(my + 1, N_DEV)

        barrier_sem = pltpu.get_barrier_semaphore()
        pl.semaphore_signal(
            barrier_sem, inc=1, device_id=(left,),
            device_id_type=pl.DeviceIdType.MESH,
        )
        pl.semaphore_signal(
            barrier_sem, inc=1, device_id=(right,),
            device_id_type=pl.DeviceIdType.MESH,
        )
        pl.semaphore_wait(barrier_sem, 2)

        local = pltpu.make_async_copy(
            x_ref, out_ref.at[pl.ds(my * m, m)], copy_sem
        )
        local.start()

        for h in range(N_DEV - 1):
            origin_r = lax.rem(my - h + N_DEV, N_DEV)
            origin_l = lax.rem(my + h, N_DEV)
            if h == 0:
                src_r = x_ref.at[pl.ds(0, mh)]
                src_l = x_ref.at[pl.ds(mh, mh)]
            else:
                src_r = out_ref.at[pl.ds(origin_r * m, mh)]
                src_l = out_ref.at[pl.ds(origin_l * m + mh, mh)]
            rdma_r = pltpu.make_async_remote_copy(
                src_ref=src_r,
                dst_ref=out_ref.at[pl.ds(origin_r * m, mh)],
                send_sem=send_r.at[h],
                recv_sem=recv_r.at[h],
                device_id=(right,),
                device_id_type=pl.DeviceIdType.MESH,
            )
            rdma_l = pltpu.make_async_remote_copy(
                src_ref=src_l,
                dst_ref=out_ref.at[pl.ds(origin_l * m + mh, mh)],
                send_sem=send_l.at[h],
                recv_sem=recv_l.at[h],
                device_id=(left,),
                device_id_type=pl.DeviceIdType.MESH,
            )
            rdma_r.start()
            rdma_l.start()
            rdma_r.wait()
            rdma_l.wait()

        local.wait()

    return pl.pallas_call(
        body,
        out_shape=jax.ShapeDtypeStruct((N_DEV * m, n), x.dtype),
        in_specs=[pl.BlockSpec(memory_space=pltpu.MemorySpace.HBM)],
        out_specs=pl.BlockSpec(memory_space=pltpu.MemorySpace.HBM),
        scratch_shapes=[
            pltpu.SemaphoreType.DMA,
            pltpu.SemaphoreType.DMA((N_DEV - 1,)),
            pltpu.SemaphoreType.DMA((N_DEV - 1,)),
            pltpu.SemaphoreType.DMA((N_DEV - 1,)),
            pltpu.SemaphoreType.DMA((N_DEV - 1,)),
        ],
        compiler_params=pltpu.CompilerParams(collective_id=0),
    )(x)


# device time: 1438910 ns/iter; 2.5916x vs baseline; 1.0099x over previous
import jax
import jax.numpy as jnp
from jax import lax
from jax.experimental import pallas as pl
from jax.experimental.pallas import tpu as pltpu

N_DEV = 8
NSUB = 4


def kernel(x):
    m, n = x.shape

    mh = m // 2
    msub = mh // NSUB

    def body(x_ref, out_ref, copy_sem, send_r, recv_r, send_l, recv_l):
        my = lax.axis_index("i")
        left = lax.rem(my + N_DEV - 1, N_DEV)
        right = lax.rem(my + 1, N_DEV)

        barrier_sem = pltpu.get_barrier_semaphore()
        pl.semaphore_signal(
            barrier_sem, inc=1, device_id=(left,),
            device_id_type=pl.DeviceIdType.MESH,
        )
        pl.semaphore_signal(
            barrier_sem, inc=1, device_id=(right,),
            device_id_type=pl.DeviceIdType.MESH,
        )
        pl.semaphore_wait(barrier_sem, 2)

        local = pltpu.make_async_copy(
            x_ref, out_ref.at[pl.ds(my * m, m)], copy_sem
        )
        local.start()

        rdmas_r = {}
        rdmas_l = {}
        for h in range(N_DEV - 1):
            origin_r = lax.rem(my - h + N_DEV, N_DEV)
            origin_l = lax.rem(my + h, N_DEV)
            for s in range(NSUB):
                off_r = origin_r * m + s * msub
                off_l = origin_l * m + mh + s * msub
                if h > 0:
                    rdmas_r[(h - 1, s)].wait_recv()
                    rdmas_l[(h - 1, s)].wait_recv()
                if h == 0:
                    src_r = x_ref.at[pl.ds(s * msub, msub)]
                    src_l = x_ref.at[pl.ds(mh + s * msub, msub)]
                else:
                    src_r = out_ref.at[pl.ds(off_r, msub)]
                    src_l = out_ref.at[pl.ds(off_l, msub)]
                rdma_r = pltpu.make_async_remote_copy(
                    src_ref=src_r,
                    dst_ref=out_ref.at[pl.ds(off_r, msub)],
                    send_sem=send_r.at[h, s],
                    recv_sem=recv_r.at[h, s],
                    device_id=(right,),
                    device_id_type=pl.DeviceIdType.MESH,
                )
                rdma_l = pltpu.make_async_remote_copy(
                    src_ref=src_l,
                    dst_ref=out_ref.at[pl.ds(off_l, msub)],
                    send_sem=send_l.at[h, s],
                    recv_sem=recv_l.at[h, s],
                    device_id=(left,),
                    device_id_type=pl.DeviceIdType.MESH,
                )
                rdma_r.start()
                rdma_l.start()
                rdmas_r[(h, s)] = rdma_r
                rdmas_l[(h, s)] = rdma_l

        for s in range(NSUB):
            rdmas_r[(N_DEV - 2, s)].wait_recv()
            rdmas_l[(N_DEV - 2, s)].wait_recv()
        for h in range(N_DEV - 1):
            for s in range(NSUB):
                rdmas_r[(h, s)].wait_send()
                rdmas_l[(h, s)].wait_send()

        local.wait()

    return pl.pallas_call(
        body,
        out_shape=jax.ShapeDtypeStruct((N_DEV * m, n), x.dtype),
        in_specs=[pl.BlockSpec(memory_space=pltpu.MemorySpace.HBM)],
        out_specs=pl.BlockSpec(memory_space=pltpu.MemorySpace.HBM),
        scratch_shapes=[
            pltpu.SemaphoreType.DMA,
            pltpu.SemaphoreType.DMA((N_DEV - 1, NSUB)),
            pltpu.SemaphoreType.DMA((N_DEV - 1, NSUB)),
            pltpu.SemaphoreType.DMA((N_DEV - 1, NSUB)),
            pltpu.SemaphoreType.DMA((N_DEV - 1, NSUB)),
        ],
        compiler_params=pltpu.CompilerParams(collective_id=0),
    )(x)


# device time: 1438039 ns/iter; 2.5932x vs baseline; 1.0006x over previous
import jax
import jax.numpy as jnp
from jax import lax
from jax.experimental import pallas as pl
from jax.experimental.pallas import tpu as pltpu

N_DEV = 8
NSUB = 4


def kernel(x):
    m, n = x.shape

    mh = m // 2
    msub = mh // NSUB

    def body(x_ref, out_ref, copy_sem, send_r, recv_r, send_l, recv_l):
        my = lax.axis_index("i")

        def ring2mesh(t):
            return jnp.where(t < 4, t, 11 - t)

        rp = ring2mesh(my)
        left = ring2mesh(lax.rem(rp + N_DEV - 1, N_DEV))
        right = ring2mesh(lax.rem(rp + 1, N_DEV))

        barrier_sem = pltpu.get_barrier_semaphore()
        pl.semaphore_signal(
            barrier_sem, inc=1, device_id=(left,),
            device_id_type=pl.DeviceIdType.MESH,
        )
        pl.semaphore_signal(
            barrier_sem, inc=1, device_id=(right,),
            device_id_type=pl.DeviceIdType.MESH,
        )
        pl.semaphore_wait(barrier_sem, 2)

        local = pltpu.make_async_copy(
            x_ref, out_ref.at[pl.ds(my * m, m)], copy_sem
        )
        local.start()

        rdmas_r = {}
        rdmas_l = {}
        for h in range(N_DEV - 1):
            origin_r = ring2mesh(lax.rem(rp - h + N_DEV, N_DEV))
            origin_l = ring2mesh(lax.rem(rp + h, N_DEV))
            for s in range(NSUB):
                off_r = origin_r * m + s * msub
                off_l = origin_l * m + mh + s * msub
                if h > 0:
                    rdmas_r[(h - 1, s)].wait_recv()
                    rdmas_l[(h - 1, s)].wait_recv()
                if h == 0:
                    src_r = x_ref.at[pl.ds(s * msub, msub)]
                    src_l = x_ref.at[pl.ds(mh + s * msub, msub)]
                else:
                    src_r = out_ref.at[pl.ds(off_r, msub)]
                    src_l = out_ref.at[pl.ds(off_l, msub)]
                rdma_r = pltpu.make_async_remote_copy(
                    src_ref=src_r,
                    dst_ref=out_ref.at[pl.ds(off_r, msub)],
                    send_sem=send_r.at[h, s],
                    recv_sem=recv_r.at[h, s],
                    device_id=(right,),
                    device_id_type=pl.DeviceIdType.MESH,
                )
                rdma_l = pltpu.make_async_remote_copy(
                    src_ref=src_l,
                    dst_ref=out_ref.at[pl.ds(off_l, msub)],
                    send_sem=send_l.at[h, s],
                    recv_sem=recv_l.at[h, s],
                    device_id=(left,),
                    device_id_type=pl.DeviceIdType.MESH,
                )
                rdma_r.start()
                rdma_l.start()
                rdmas_r[(h, s)] = rdma_r
                rdmas_l[(h, s)] = rdma_l

        for s in range(NSUB):
            rdmas_r[(N_DEV - 2, s)].wait_recv()
            rdmas_l[(N_DEV - 2, s)].wait_recv()
        for h in range(N_DEV - 1):
            for s in range(NSUB):
                rdmas_r[(h, s)].wait_send()
                rdmas_l[(h, s)].wait_send()

        local.wait()

    return pl.pallas_call(
        body,
        out_shape=jax.ShapeDtypeStruct((N_DEV * m, n), x.dtype),
        in_specs=[pl.BlockSpec(memory_space=pltpu.MemorySpace.HBM)],
        out_specs=pl.BlockSpec(memory_space=pltpu.MemorySpace.HBM),
        scratch_shapes=[
            pltpu.SemaphoreType.DMA,
            pltpu.SemaphoreType.DMA((N_DEV - 1, NSUB)),
            pltpu.SemaphoreType.DMA((N_DEV - 1, NSUB)),
            pltpu.SemaphoreType.DMA((N_DEV - 1, NSUB)),
            pltpu.SemaphoreType.DMA((N_DEV - 1, NSUB)),
        ],
        compiler_params=pltpu.CompilerParams(collective_id=0),
    )(x)


# device time: 1196105 ns/iter; 3.1177x vs baseline; 1.2023x over previous
import jax
import jax.numpy as jnp
from jax import lax
from jax.experimental import pallas as pl
from jax.experimental.pallas import tpu as pltpu

N_DEV = 8

PART_OFFS = ((0, 2728), (2728, 2728), (5456, 2736))
N_SENDS = 3 + 6 + 12


def kernel(x):
    m, n = x.shape

    def body(x_ref, out_ref, copy_sem, send_sems, recv_sems):
        p = lax.axis_index("i")
        z = lax.div(p, 4)
        q = lax.rem(p, 4)
        cx = jnp.where((q == 1) | (q == 2), 1, 0)
        cy = jnp.where(q >= 2, 1, 0)

        def mesh_idx(ax, ay, az):
            return ax + ay * (3 - 2 * ax) + 4 * az

        xn = mesh_idx(1 - cx, cy, z)
        yn = mesh_idx(cx, 1 - cy, z)
        zn = mesh_idx(cx, cy, 1 - z)
        xyn = mesh_idx(1 - cx, 1 - cy, z)
        xzn = mesh_idx(1 - cx, cy, 1 - z)
        yzn = mesh_idx(cx, 1 - cy, 1 - z)

        barrier_sem = pltpu.get_barrier_semaphore()
        for nbr in (xn, yn, zn):
            pl.semaphore_signal(
                barrier_sem, inc=1, device_id=(nbr,),
                device_id_type=pl.DeviceIdType.MESH,
            )
        pl.semaphore_wait(barrier_sem, 3)

        local = pltpu.make_async_copy(
            x_ref, out_ref.at[pl.ds(p * m, m)], copy_sem
        )
        local.start()

        sends = []

        def send(part, origin, dest, own):
            k = len(sends)
            poff, plen = PART_OFFS[part]
            if own:
                src = x_ref.at[pl.ds(poff, plen)]
            else:
                src = out_ref.at[pl.ds(origin * m + poff, plen)]
            rdma = pltpu.make_async_remote_copy(
                src_ref=src,
                dst_ref=out_ref.at[pl.ds(origin * m + poff, plen)],
                send_sem=send_sems.at[k],
                recv_sem=recv_sems.at[k],
                device_id=(dest,),
                device_id_type=pl.DeviceIdType.MESH,
            )
            rdma.start()
            sends.append(rdma)
            return rdma

        s1 = [
            send(0, p, xn, True),
            send(1, p, yn, True),
            send(2, p, zn, True),
        ]
        for r in s1:
            r.wait()

        s2 = [
            send(0, p, yn, True), send(0, xn, yn, False),
            send(1, p, zn, True), send(1, yn, zn, False),
            send(2, p, xn, True), send(2, zn, xn, False),
        ]
        for r in s2:
            r.wait()

        s3 = [
            send(0, p, zn, True),
            send(0, xn, zn, False),
            send(0, yn, zn, False),
            send(0, xyn, zn, False),
            send(1, p, xn, True),
            send(1, yn, xn, False),
            send(1, zn, xn, False),
            send(1, yzn, xn, False),
            send(2, p, yn, True),
            send(2, zn, yn, False),
            send(2, xn, yn, False),
            send(2, xzn, yn, False),
        ]
        for r in s3:
            r.wait()

        local.wait()

    return pl.pallas_call(
        body,
        out_shape=jax.ShapeDtypeStruct((N_DEV * m, n), x.dtype),
        in_specs=[pl.BlockSpec(memory_space=pl.ANY)],
        out_specs=pl.BlockSpec(memory_space=pl.ANY),
        scratch_shapes=[
            pltpu.SemaphoreType.DMA,
            pltpu.SemaphoreType.DMA((N_SENDS,)),
            pltpu.SemaphoreType.DMA((N_SENDS,)),
        ],
        compiler_params=pltpu.CompilerParams(collective_id=0),
    )(x)
